# baseline (speedup 1.0000x reference)
import jax
import jax.numpy as jnp
from jax import lax
from jax.experimental import pallas as pl
from jax.experimental.pallas import tpu as pltpu
from jax.experimental.pallas import tpu_sc as plsc

NUM_CLASSES_ = 100000
FEATURE_DIM_ = 64
BATCH_ = 16384

NC = 2
NS = 16
L = 16

EPT = BATCH_ // NS
CH = 128
NCHUNK = EPT // CH
W = FEATURE_DIM_ + L
NP = 3
PS = 16768
RPT = PS // NS
HALF = NP * PS
ACC_T = NC * HALF
ACC_ROWS = PS + 1
NBLK = 50
RB = NUM_CLASSES_ // NBLK


def _sc_kernel(inputs_hbm, labels_hbm,
               acc_hbm, loss_hbm,
               lbl_v, sb_v, idx_v, zb_v, lp_v, acc_sh, sem):
    c = lax.axis_index("c")
    s = lax.axis_index("s")
    base_e = pl.multiple_of(s * EPT, CH)

    ones16 = jnp.full((L,), 1.0, jnp.float32)
    zeros16 = jnp.zeros((L,), jnp.float32)

    pltpu.sync_copy(labels_hbm.at[pl.ds(base_e, EPT)], lbl_v)

    def _zb(r, _):
        for q in range(W // L):
            zb_v[r, pl.ds(q * L, L)] = zeros16
        sb_v[r, pl.ds(FEATURE_DIM_, L)] = ones16
        return 0
    lax.fori_loop(0, CH, _zb, 0)

    accs = (zeros16, zeros16, zeros16, zeros16)
    lo = pl.multiple_of(s * RPT, 8)
    for p in range(NP):
        base_cls = c * HALF + p * PS

        nfull, rem = divmod(RPT, CH)
        for k in range(nfull):
            pltpu.sync_copy(zb_v, acc_sh.at[pl.ds(lo + k * CH, CH)])
        if rem:
            pltpu.sync_copy(zb_v.at[pl.ds(0, rem), :],
                            acc_sh.at[pl.ds(lo + nfull * CH, rem)])

        def _idx(k, _):
            v = lbl_v[pl.ds(k * L, L)]
            rel = v - base_cls
            ok = (rel >= 0) & (rel < PS)
            idx_v[k // (CH // L), pl.ds((k % (CH // L)) * L, L)] = (
                jnp.where(ok, rel, PS))
            return 0
        lax.fori_loop(0, EPT // L, _idx, 0)

        plsc.subcore_barrier()

        for j in range(NCHUNK):
            pltpu.sync_copy(
                inputs_hbm.at[pl.ds(base_e + j * CH, CH), :],
                sb_v.at[:, pl.ds(0, FEATURE_DIM_)])
            if p == 0:
                def _row(r, acc):
                    out = []
                    for q in range(FEATURE_DIM_ // L):
                        v = sb_v[r, pl.ds(q * L, L)]
                        out.append(acc[q] + v * v)
                    return tuple(out)
                accs = lax.fori_loop(0, CH, _row, accs)
            pltpu.sync_copy(sb_v, acc_sh.at[idx_v.at[j]], add=True)

        plsc.subcore_barrier()

        pltpu.sync_copy(acc_sh.at[pl.ds(lo, RPT)],
                        acc_hbm.at[pl.ds(pl.multiple_of(base_cls + lo, 8),
                                         RPT), pl.ds(0, W)])

        plsc.subcore_barrier()

    lp_v[0, :] = accs[0] + accs[1] + accs[2] + accs[3]
    wid = c * NS + s
    pltpu.sync_copy(lp_v, loss_hbm.at[wid])


def _tc_body(cen_ref, acc_ref, lp_ref, out_ref, loss_ref):
    acc = acc_ref[...]
    xsum = acc[:, :FEATURE_DIM_]
    cnt = acc[:, FEATURE_DIM_:FEATURE_DIM_ + 1]
    cen = cen_ref[...]
    num = cnt * cen - xsum
    out_ref[...] = cen - num / (cnt + 1.0)

    part = jnp.sum((cnt * cen - 2.0 * xsum) * cen)

    i = pl.program_id(0)

    @pl.when(i == 0)
    def _():
        loss_ref[...] = jnp.reshape(jnp.sum(lp_ref[...]) * 0.5, (1, 1))

    loss_ref[...] = loss_ref[...] + jnp.reshape(part, (1, 1))

    @pl.when(i == NBLK - 1)
    def _():
        loss_ref[...] = loss_ref[...] * (1.0 / BATCH_)


def kernel(inputs, labels, centers):
    labels = jnp.reshape(labels, (-1,)).astype(jnp.int32)

    mesh = plsc.VectorSubcoreMesh(core_axis_name="c", subcore_axis_name="s")
    sc = pl.kernel(
        _sc_kernel,
        out_type=(
            jax.ShapeDtypeStruct((ACC_T, 128), jnp.float32),
            jax.ShapeDtypeStruct((NC * NS, 1, L), jnp.float32),
        ),
        mesh=mesh,
        compiler_params=pltpu.CompilerParams(use_tc_tiling_on_sc=False),
        scratch_types=[
            pltpu.VMEM((EPT,), jnp.int32),
            pltpu.VMEM((CH, W), jnp.float32),
            pltpu.VMEM((NCHUNK, CH), jnp.int32),
            pltpu.VMEM((CH, W), jnp.float32),
            pltpu.VMEM((1, L), jnp.float32),
            pltpu.VMEM_SHARED((ACC_ROWS, W), jnp.float32),
            pltpu.SemaphoreType.DMA,
        ],
    )
    acc, loss_part = sc(inputs, labels)
    loss_part = jnp.reshape(loss_part, (NC * NS, L))

    new_centers, loss = pl.pallas_call(
        _tc_body,
        grid=(NBLK,),
        in_specs=[
            pl.BlockSpec((RB, FEATURE_DIM_), lambda i: (i, 0)),
            pl.BlockSpec((RB, 128), lambda i: (i, 0)),
            pl.BlockSpec((NC * NS, L), lambda i: (0, 0)),
        ],
        out_specs=[
            pl.BlockSpec((RB, FEATURE_DIM_), lambda i: (i, 0)),
            pl.BlockSpec((1, 1), lambda i: (0, 0)),
        ],
        out_shape=[
            jax.ShapeDtypeStruct((NUM_CLASSES_, FEATURE_DIM_), jnp.float32),
            jax.ShapeDtypeStruct((1, 1), jnp.float32),
        ],
        input_output_aliases={0: 0},
    )(centers, acc, loss_part)

    return inputs, new_centers, jnp.reshape(loss, ())


# speedup vs baseline: 1.2708x; 1.2708x over previous
import jax
import jax.numpy as jnp
from jax import lax
from jax.experimental import pallas as pl
from jax.experimental.pallas import tpu as pltpu
from jax.experimental.pallas import tpu_sc as plsc

NUM_CLASSES_ = 100000
FEATURE_DIM_ = 64
BATCH_ = 16384

NC = 2
NS = 16
L = 16

EPT = BATCH_ // NS
CH = 128
NCHUNK = EPT // CH
W = FEATURE_DIM_ + L
NP = 3
PS = 16768
RPT = PS // NS
HALF = NP * PS
ACC_T = NC * HALF
ACC_ROWS = PS + 1
CB = 2048
NBLK = -(-NUM_CLASSES_ // CB)


def _sc_kernel(inputs_hbm, labels_hbm,
               acc_hbm, loss_hbm,
               lbl_v, sb_v, idx_v, zb_v, lp_v, acc_sh, sem):
    c = lax.axis_index("c")
    s = lax.axis_index("s")
    base_e = pl.multiple_of(s * EPT, CH)

    ones16 = jnp.full((L,), 1.0, jnp.float32)
    zeros16 = jnp.zeros((L,), jnp.float32)

    pltpu.sync_copy(labels_hbm.at[pl.ds(base_e, EPT)], lbl_v)

    def _zb(r, _):
        for q in range(W // L):
            zb_v[r, pl.ds(q * L, L)] = zeros16
        sb_v[r, pl.ds(FEATURE_DIM_, L)] = ones16
        return 0
    lax.fori_loop(0, CH, _zb, 0)

    accs = (zeros16, zeros16, zeros16, zeros16)
    lo = pl.multiple_of(s * RPT, 8)
    for p in range(NP):
        base_cls = c * HALF + p * PS

        nfull, rem = divmod(RPT, CH)
        for k in range(nfull):
            pltpu.sync_copy(zb_v, acc_sh.at[pl.ds(lo + k * CH, CH)])
        if rem:
            pltpu.sync_copy(zb_v.at[pl.ds(0, rem), :],
                            acc_sh.at[pl.ds(lo + nfull * CH, rem)])

        def _idx(k, _):
            v = lbl_v[pl.ds(k * L, L)]
            rel = v - base_cls
            ok = (rel >= 0) & (rel < PS)
            idx_v[k // (CH // L), pl.ds((k % (CH // L)) * L, L)] = (
                jnp.where(ok, rel, PS))
            return 0
        lax.fori_loop(0, EPT // L, _idx, 0)

        plsc.subcore_barrier()

        for j in range(NCHUNK):
            pltpu.sync_copy(
                inputs_hbm.at[pl.ds(base_e + j * CH, CH), :],
                sb_v.at[:, pl.ds(0, FEATURE_DIM_)])
            if p == 0:
                def _row(r, acc):
                    out = []
                    for q in range(FEATURE_DIM_ // L):
                        v = sb_v[r, pl.ds(q * L, L)]
                        out.append(acc[q] + v * v)
                    return tuple(out)
                accs = lax.fori_loop(0, CH, _row, accs)
            pltpu.sync_copy(sb_v, acc_sh.at[idx_v.at[j]], add=True)

        plsc.subcore_barrier()

        pltpu.sync_copy(acc_sh.at[pl.ds(lo, RPT)],
                        acc_hbm.at[pl.ds(pl.multiple_of(base_cls + lo, 8),
                                         RPT), pl.ds(0, W)])

        plsc.subcore_barrier()

    lp_v[0, :] = accs[0] + accs[1] + accs[2] + accs[3]
    wid = c * NS + s
    pltpu.sync_copy(lp_v, loss_hbm.at[wid])


def _tc_body(cen_ref, acc_ref, lp_ref, out_ref, loss_ref):
    acct = jnp.transpose(acc_ref[...])
    xsum = acct[:FEATURE_DIM_, :]
    cnt = acct[FEATURE_DIM_:FEATURE_DIM_ + 1, :]
    cen = cen_ref[...]
    num = cnt * cen - xsum
    out_ref[...] = cen - num / (cnt + 1.0)

    i = pl.program_id(0)

    col = i * CB + jax.lax.broadcasted_iota(jnp.int32, (FEATURE_DIM_, CB), 1)
    term = jnp.where(col < NUM_CLASSES_, (cnt * cen - 2.0 * xsum) * cen, 0.0)
    part = jnp.sum(term)

    @pl.when(i == 0)
    def _():
        loss_ref[...] = jnp.reshape(jnp.sum(lp_ref[...]) * 0.5, (1, 1))

    loss_ref[...] = loss_ref[...] + jnp.reshape(part, (1, 1))

    @pl.when(i == NBLK - 1)
    def _():
        loss_ref[...] = loss_ref[...] * (1.0 / BATCH_)


def kernel(inputs, labels, centers):
    labels = jnp.reshape(labels, (-1,)).astype(jnp.int32)

    mesh = plsc.VectorSubcoreMesh(core_axis_name="c", subcore_axis_name="s")
    sc = pl.kernel(
        _sc_kernel,
        out_type=(
            jax.ShapeDtypeStruct((ACC_T, 128), jnp.float32),
            jax.ShapeDtypeStruct((NC * NS, 1, L), jnp.float32),
        ),
        mesh=mesh,
        compiler_params=pltpu.CompilerParams(use_tc_tiling_on_sc=False),
        scratch_types=[
            pltpu.VMEM((EPT,), jnp.int32),
            pltpu.VMEM((CH, W), jnp.float32),
            pltpu.VMEM((NCHUNK, CH), jnp.int32),
            pltpu.VMEM((CH, W), jnp.float32),
            pltpu.VMEM((1, L), jnp.float32),
            pltpu.VMEM_SHARED((ACC_ROWS, W), jnp.float32),
            pltpu.SemaphoreType.DMA,
        ],
    )
    acc, loss_part = sc(inputs, labels)
    loss_part = jnp.reshape(loss_part, (NC * NS, L))

    cen_t = jnp.transpose(centers)
    new_t, loss = pl.pallas_call(
        _tc_body,
        grid=(NBLK,),
        in_specs=[
            pl.BlockSpec((FEATURE_DIM_, CB), lambda i: (0, i)),
            pl.BlockSpec((CB, 128), lambda i: (i, 0)),
            pl.BlockSpec((NC * NS, L), lambda i: (0, 0)),
        ],
        out_specs=[
            pl.BlockSpec((FEATURE_DIM_, CB), lambda i: (0, i)),
            pl.BlockSpec((1, 1), lambda i: (0, 0)),
        ],
        out_shape=[
            jax.ShapeDtypeStruct((FEATURE_DIM_, NUM_CLASSES_), jnp.float32),
            jax.ShapeDtypeStruct((1, 1), jnp.float32),
        ],
    )(cen_t, acc, loss_part)

    return inputs, jnp.transpose(new_t), jnp.reshape(loss, ())


# speedup vs baseline: 1.3700x; 1.0781x over previous
import functools

import jax
import jax.numpy as jnp
from jax import lax
from jax.experimental import pallas as pl
from jax.experimental.pallas import tpu as pltpu
from jax.experimental.pallas import tpu_sc as plsc

NUM_CLASSES_ = 100000
FEATURE_DIM_ = 64
BATCH_ = 16384

NC = 2
NS = 16
L = 16

EPT = BATCH_ // NS
CH = 128
NCHUNK = EPT // CH
W = FEATURE_DIM_ + L
CB = 2048
NPASS = 3
JMAX = 9
PS = JMAX * CB
RPT = PS // NS
NBTOT = -(-NUM_CLASSES_ // CB)


def _nblocks(c, p):
    n = 0
    while (2 * (3 * n + p) + c) * CB < NUM_CLASSES_:
        n += 1
    return n


def _sc_pass(p, inputs_hbm, labels_hbm, *args):
    if p == 0:
        acc_hbm, loss_hbm, lbl_v, sb_v, idx_v, zb_v, lp_v, acc_sh, sem = args
    else:
        prev_hbm, acc_hbm, lbl_v, sb_v, idx_v, zb_v, acc_sh, sem = args
    c = lax.axis_index("c")
    s = lax.axis_index("s")
    base_e = pl.multiple_of(s * EPT, CH)

    ones16 = jnp.full((L,), 1.0, jnp.float32)
    zeros16 = jnp.zeros((L,), jnp.float32)

    pltpu.sync_copy(labels_hbm.at[pl.ds(base_e, EPT)], lbl_v)

    def _zb(r, _):
        for q in range(W // L):
            zb_v[r, pl.ds(q * L, L)] = zeros16
        sb_v[r, pl.ds(FEATURE_DIM_, L)] = ones16
        return 0
    lax.fori_loop(0, CH, _zb, 0)

    lo = pl.multiple_of(s * RPT, 8)

    for k in range(RPT // CH):
        pltpu.sync_copy(zb_v, acc_sh.at[pl.ds(lo + k * CH, CH)])

    def _idx(k, _):
        v = lbl_v[pl.ds(k * L, L)]
        b = lax.shift_right_logical(v, 11)
        n = lax.shift_right_logical(b, 1)
        j = lax.shift_right_logical(n * 21846, 16)
        pm = n - 3 * j
        cm = b & 1
        ok = (cm == c) & (pm == p)
        rel = j * CB + (v & (CB - 1))
        idx_v[k // (CH // L), pl.ds((k % (CH // L)) * L, L)] = (
            jnp.where(ok, rel, PS))
        return 0
    lax.fori_loop(0, EPT // L, _idx, 0)

    plsc.subcore_barrier()

    if p == 0:
        accs = (zeros16, zeros16, zeros16, zeros16)
    for j in range(NCHUNK):
        pltpu.sync_copy(
            inputs_hbm.at[pl.ds(base_e + j * CH, CH), :],
            sb_v.at[:, pl.ds(0, FEATURE_DIM_)])
        if p == 0:
            def _row(r, acc):
                out = []
                for q in range(FEATURE_DIM_ // L):
                    v = sb_v[r, pl.ds(q * L, L)]
                    out.append(acc[q] + v * v)
                return tuple(out)
            accs = lax.fori_loop(0, CH, _row, accs)
        pltpu.sync_copy(sb_v, acc_sh.at[idx_v.at[j]], add=True)

    plsc.subcore_barrier()

    pltpu.sync_copy(acc_sh.at[pl.ds(lo, RPT)],
                    acc_hbm.at[pl.ds(pl.multiple_of(c * PS + lo, 8),
                                     RPT), pl.ds(0, W)])

    if p == 0:
        lp_v[0, :] = accs[0] + accs[1] + accs[2] + accs[3]
        wid = c * NS + s
        pltpu.sync_copy(lp_v, loss_hbm.at[wid])


def _tc_body(p, nc0, cen_ref, acc_ref, lp_ref, out_ref, loss_ref):
    i = pl.program_id(0)
    ci = jnp.where(i >= nc0, 1, 0)
    k = i - ci * nc0
    b = 2 * (3 * k + p) + ci

    acct = jnp.transpose(acc_ref[...])
    xsum = acct[:FEATURE_DIM_, :]
    cnt = acct[FEATURE_DIM_:FEATURE_DIM_ + 1, :]
    cen = cen_ref[...]
    num = cnt * cen - xsum
    out_ref[...] = cen - num / (cnt + 1.0)

    col = b * CB + jax.lax.broadcasted_iota(jnp.int32, (FEATURE_DIM_, CB), 1)
    term = jnp.where(col < NUM_CLASSES_, (cnt * cen - 2.0 * xsum) * cen, 0.0)
    part = jnp.sum(term)

    @pl.when(i == 0)
    def _():
        if p == 0:
            loss_ref[...] = jnp.reshape(jnp.sum(lp_ref[...]) * 0.5, (1, 1))
        else:
            loss_ref[...] = jnp.zeros((1, 1), jnp.float32)

    loss_ref[...] = loss_ref[...] + jnp.reshape(part, (1, 1))


def kernel(inputs, labels, centers):
    labels = jnp.reshape(labels, (-1,)).astype(jnp.int32)

    mesh = plsc.VectorSubcoreMesh(core_axis_name="c", subcore_axis_name="s")
    common_scratch = [
        pltpu.VMEM((EPT,), jnp.int32),
        pltpu.VMEM((CH, W), jnp.float32),
        pltpu.VMEM((NCHUNK, CH), jnp.int32),
        pltpu.VMEM((CH, W), jnp.float32),
    ]
    accs = []
    loss_part = None
    for p in range(NPASS):
        if p == 0:
            out_type = (
                jax.ShapeDtypeStruct((NC * PS, 128), jnp.float32),
                jax.ShapeDtypeStruct((NC * NS, 1, L), jnp.float32),
            )
            scratch = common_scratch + [
                pltpu.VMEM((1, L), jnp.float32),
                pltpu.VMEM_SHARED((PS + 1, W), jnp.float32),
                pltpu.SemaphoreType.DMA,
            ]
        else:
            out_type = jax.ShapeDtypeStruct((NC * PS, 128), jnp.float32)
            scratch = common_scratch + [
                pltpu.VMEM_SHARED((PS + 1, W), jnp.float32),
                pltpu.SemaphoreType.DMA,
            ]
        sc = pl.kernel(
            functools.partial(_sc_pass, p),
            out_type=out_type,
            mesh=mesh,
            compiler_params=pltpu.CompilerParams(use_tc_tiling_on_sc=False),
            scratch_types=scratch,
            name=f"sc_pass{p}",
        )
        if p == 0:
            acc, loss_part = sc(inputs, labels)
        else:
            acc = sc(inputs, labels, accs[p - 1])
        accs.append(acc)
    loss_part = jnp.reshape(loss_part, (NC * NS, L))

    cen_t = jnp.transpose(centers)
    out_t = None
    losses = []
    for p in range(NPASS):
        nc0 = _nblocks(0, p)
        nblk = nc0 + _nblocks(1, p)

        def cen_map(i, nc0=nc0, p=p):
            ci = jnp.where(i >= nc0, 1, 0)
            k = i - ci * nc0
            return (0, 2 * (3 * k + p) + ci)

        def acc_map(i, nc0=nc0):
            ci = jnp.where(i >= nc0, 1, 0)
            k = i - ci * nc0
            return (ci * JMAX + k, 0)

        out_shape = [
            jax.ShapeDtypeStruct((FEATURE_DIM_, NUM_CLASSES_), jnp.float32),
            jax.ShapeDtypeStruct((1, 1), jnp.float32),
        ]
        alias = {} if out_t is None else {0: 0}
        operands = [cen_t if out_t is None else out_t, accs[p], loss_part]
        out_t, lp = pl.pallas_call(
            functools.partial(_tc_body, p, nc0),
            grid=(nblk,),
            in_specs=[
                pl.BlockSpec((FEATURE_DIM_, CB), cen_map),
                pl.BlockSpec((CB, 128), acc_map),
                pl.BlockSpec((NC * NS, L), lambda i: (0, 0)),
            ],
            out_specs=[
                pl.BlockSpec((FEATURE_DIM_, CB), cen_map),
                pl.BlockSpec((1, 1), lambda i: (0, 0)),
            ],
            out_shape=out_shape,
            input_output_aliases=alias,
        )(*operands)
        losses.append(lp)

    loss = (losses[0] + losses[1] + losses[2]) * (1.0 / BATCH_)
    return inputs, jnp.transpose(out_t), jnp.reshape(loss, ())
